# Initial kernel scaffold; baseline (speedup 1.0000x reference)
import jax
import jax.numpy as jnp
from jax.experimental import pallas as pl


def kernel(x, expert_ids, expert_weights):
    raise NotImplementedError("write your pallas kernel here")


# speedup vs baseline: 3.1374x; 3.1374x over previous
import jax
import jax.numpy as jnp
from jax import lax
from jax.experimental import pallas as pl
from jax.experimental.pallas import tpu as pltpu
from jax.experimental.pallas import tpu_sc as plsc

_E = 16
_CAP = 1280
_T = 16384
_D = 2048
_HALF = _CAP // 2
_CHUNK = 16
_NCHUNK = _HALF // _CHUNK
_NVEC = _T // 16
_LIST = _CAP + 32


def _body(x_hbm, ids_hbm, ew_hbm, z_hbm,
          out_x, out_w, out_ti, out_d,
          ids_v, list_v, idxc_v, buf0_v, buf1_v, zbuf_v, wbuf_v, tibuf_v,
          dvec_v, kept_smem, gsem0, gsem1, wsem0, wsem1, zsem, esem):
  e = lax.axis_index("s")
  half = lax.axis_index("c")
  iota = lax.iota(jnp.int32, 16)

  kept_smem[0] = 0
  plsc.subcore_barrier()

  pltpu.sync_copy(ids_hbm, ids_v)
  pltpu.sync_copy(z_hbm, zbuf_v)

  def scan_step(i, cnt):
    ids16 = ids_v[pl.ds(i * 16, 16)]
    m = ids16 == e
    mi = m.astype(jnp.int32)
    cs = plsc.cumsum(mi)
    toks = iota + i * 16
    rank = cnt + cs - 1
    dest = jnp.where(jnp.logical_and(m, rank < _CAP), rank, _CAP)
    plsc.store_scatter(list_v, [dest], toks)
    return cnt + jnp.sum(mi)

  count = lax.fori_loop(0, _NVEC, scan_step, jnp.int32(0))

  n_kept = jnp.minimum(count, _CAP)
  v = jnp.clip(n_kept - half * _HALF, 0, _HALF)

  plsc.fetch_and_add(kept_smem.at[0], n_kept, subcore_id=0)
  plsc.subcore_barrier()

  @pl.when(jnp.logical_and(e == 0, half == 0))
  def _():
    kept = kept_smem[0]
    dvec_v[...] = jnp.full((16,), _T - kept, jnp.int32)
    pltpu.sync_copy(dvec_v, out_d)

  def idx_step(j, _):
    jglob = iota + j * 16
    valid = jglob < v
    idx16 = list_v[pl.ds(half * _HALF + j * 16, 16)]
    idxc_v[pl.ds(j * 16, 16)] = jnp.where(valid, idx16, 0)
    tibuf_v[pl.ds(j * 16, 16)] = jnp.where(valid, idx16, -1)
    return 0

  lax.fori_loop(0, _NCHUNK, idx_step, 0)

  for g in range(_HALF // 128):
    pltpu.async_copy(ew_hbm.at[idxc_v.at[pl.ds(g * 128, 128)]],
                     wbuf_v.at[pl.ds(g * 128, 128)], esem)
  for g in range(_HALF // 128):
    pltpu.make_async_copy(ew_hbm.at[idxc_v.at[pl.ds(g * 128, 128)]],
                          wbuf_v.at[pl.ds(g * 128, 128)], esem).wait()

  def wmask_step(j, _):
    jglob = iota + j * 16
    w16 = wbuf_v[pl.ds(j * 16, 16)]
    wbuf_v[pl.ds(j * 16, 16)] = jnp.where(jglob < v, w16, 0.0)
    return 0

  lax.fori_loop(0, _NCHUNK, wmask_step, 0)

  base_out = e * _CAP + half * _HALF
  pltpu.sync_copy(wbuf_v, out_w.at[pl.ds(base_out, _HALF)])
  pltpu.sync_copy(tibuf_v, out_ti.at[pl.ds(base_out, _HALF)])

  nfull = v // _CHUNK
  rem = v - nfull * _CHUNK

  def g_issue(c, buf, sem):
    idx16 = idxc_v[pl.ds(c * _CHUNK, 16)]
    pltpu.async_copy(x_hbm.at[idx16], buf, sem)

  def g_wait(buf, sem):
    pltpu.make_async_copy(x_hbm.at[iota], buf, sem).wait()

  def w_issue(c, buf, sem):
    pltpu.async_copy(buf, out_x.at[pl.ds(base_out + c * _CHUNK, _CHUNK)], sem)

  def w_wait(buf, sem):
    pltpu.make_async_copy(buf, out_x.at[pl.ds(base_out, _CHUNK)], sem).wait()

  @pl.when(nfull > 0)
  def _():
    g_issue(0, buf0_v, gsem0)

  @pl.when(nfull > 1)
  def _():
    g_issue(1, buf1_v, gsem1)

  def pair_step(p, _):
    c0 = 2 * p
    c1 = 2 * p + 1

    g_wait(buf0_v, gsem0)
    w_issue(c0, buf0_v, wsem0)

    @pl.when(c0 + 2 < nfull)
    def _():
      w_wait(buf0_v, wsem0)
      g_issue(c0 + 2, buf0_v, gsem0)

    @pl.when(c1 < nfull)
    def _():
      g_wait(buf1_v, gsem1)
      w_issue(c1, buf1_v, wsem1)

      @pl.when(c1 + 2 < nfull)
      def _():
        w_wait(buf1_v, wsem1)
        g_issue(c1 + 2, buf1_v, gsem1)

    return 0

  npairs = (nfull + 1) // 2
  lax.fori_loop(0, npairs, pair_step, 0)

  @pl.when(nfull > 0)
  def _():
    w_wait(buf0_v, wsem0)

  @pl.when(nfull > 1)
  def _():
    w_wait(buf1_v, wsem1)

  @pl.when(rem > 0)
  def _():
    g_issue(nfull, buf0_v, gsem0)
    g_wait(buf0_v, gsem0)
    dst = base_out + nfull * _CHUNK
    pltpu.sync_copy(zbuf_v, out_x.at[pl.ds(dst, _CHUNK)])

    def row_step(r, _):
      pltpu.sync_copy(buf0_v.at[pl.ds(r, 1)], out_x.at[pl.ds(dst + r, 1)])
      return 0

    lax.fori_loop(0, rem, row_step, 0)

  zstart = nfull + jnp.where(rem > 0, 1, 0).astype(jnp.int32)

  def zwave(wv, _):
    c = zstart + wv * 4
    for k in range(4):
      @pl.when(c + k < _NCHUNK)
      def _(k=k):
        pltpu.async_copy(zbuf_v,
                         out_x.at[pl.ds(base_out + (c + k) * _CHUNK, _CHUNK)],
                         zsem)
    for k in range(4):
      @pl.when(c + k < _NCHUNK)
      def _(k=k):
        pltpu.make_async_copy(zbuf_v,
                              out_x.at[pl.ds(base_out, _CHUNK)], zsem).wait()
    return 0

  lax.fori_loop(0, 10, zwave, 0)


@jax.jit
def _dispatch(x, expert_ids, expert_weights):
  z = jnp.zeros((_CHUNK, _D), jnp.float32)
  mesh = plsc.VectorSubcoreMesh(core_axis_name="c", subcore_axis_name="s",
                                num_cores=2, num_subcores=16)
  fn = pl.kernel(
      _body,
      out_type=(
          jax.ShapeDtypeStruct((_E * _CAP, _D), jnp.float32),
          jax.ShapeDtypeStruct((_E * _CAP,), jnp.float32),
          jax.ShapeDtypeStruct((_E * _CAP,), jnp.int32),
          jax.ShapeDtypeStruct((16,), jnp.int32),
      ),
      mesh=mesh,
      compiler_params=pltpu.CompilerParams(needs_layout_passes=False),
      scratch_types=[
          pltpu.VMEM((_T,), jnp.int32),
          pltpu.VMEM((_LIST,), jnp.int32),
          pltpu.VMEM((_HALF,), jnp.int32),
          pltpu.VMEM((_CHUNK, _D), jnp.float32),
          pltpu.VMEM((_CHUNK, _D), jnp.float32),
          pltpu.VMEM((_CHUNK, _D), jnp.float32),
          pltpu.VMEM((_HALF,), jnp.float32),
          pltpu.VMEM((_HALF,), jnp.int32),
          pltpu.VMEM((16,), jnp.int32),
          pltpu.SMEM((1,), jnp.int32),
          pltpu.SemaphoreType.DMA,
          pltpu.SemaphoreType.DMA,
          pltpu.SemaphoreType.DMA,
          pltpu.SemaphoreType.DMA,
          pltpu.SemaphoreType.DMA,
          pltpu.SemaphoreType.DMA,
      ],
  )
  return fn(x, expert_ids, expert_weights, z)


def kernel(x, expert_ids, expert_weights):
  out_x, out_w, out_ti, out_d = _dispatch(x, expert_ids, expert_weights)
  return (out_x.reshape(_E, _CAP, _D),
          out_w.reshape(_E, _CAP),
          out_ti.reshape(_E, _CAP),
          out_d[0])


# speedup vs baseline: 3.4198x; 1.0900x over previous
import jax
import jax.numpy as jnp
from jax import lax
from jax.experimental import pallas as pl
from jax.experimental.pallas import tpu as pltpu
from jax.experimental.pallas import tpu_sc as plsc

_E = 16
_CAP = 1280
_T = 16384
_D = 2048
_HALF = _CAP // 2
_CHUNK = 8
_NCHUNK = _HALF // _CHUNK
_NVEC = _T // 16
_LIST = _CAP + 32


def _body(x_hbm, ids_hbm, ew_hbm, z_hbm,
          out_x, out_w, out_ti, out_d,
          ids_v, list_v, idxc_v, buf0_v, buf1_v, buf2_v, buf3_v, zbuf_v,
          wbuf_v, tibuf_v, dvec_v, kept_smem,
          gsem0, gsem1, gsem2, gsem3, wsem0, wsem1, wsem2, wsem3,
          zsem, esem, ssem):
  e = lax.axis_index("s")
  half = (lax.axis_index("s") + lax.axis_index("c")) % 2
  iota = lax.iota(jnp.int32, 16)

  kept_smem[0] = 0
  plsc.subcore_barrier()

  pltpu.sync_copy(ids_hbm, ids_v)
  pltpu.sync_copy(z_hbm, zbuf_v)

  def scan_step(i, cnt_vec):
    for u in range(2):
      base = (i * 2 + u) * 16
      ids16 = ids_v[pl.ds(base, 16)]
      m = ids16 == e
      mi = m.astype(jnp.int32)
      cs = plsc.cumsum(mi)
      tot = plsc.all_reduce_population_count(m)
      rank = cnt_vec + cs - 1
      dest = jnp.where(jnp.logical_and(m, rank < _CAP), rank, _CAP)
      plsc.store_scatter(list_v, [dest], iota + base)
      cnt_vec = cnt_vec + tot
    return cnt_vec

  cnt_vec = lax.fori_loop(0, _NVEC // 2, scan_step,
                          jnp.zeros((16,), jnp.int32))
  count = jnp.max(cnt_vec)

  n_kept = jnp.minimum(count, _CAP)
  v = jnp.clip(n_kept - half * _HALF, 0, _HALF)

  plsc.fetch_and_add(kept_smem.at[0], n_kept, subcore_id=0)
  plsc.subcore_barrier()

  @pl.when(jnp.logical_and(e == 0, half == 0))
  def _():
    kept = kept_smem[0]
    dvec_v[...] = jnp.full((16,), _T - kept, jnp.int32)
    pltpu.sync_copy(dvec_v, out_d)

  def idx_step(j, _):
    jglob = iota + j * 16
    valid = jglob < v
    idx16 = list_v[pl.ds(half * _HALF + j * 16, 16)]
    idxc_v[pl.ds(j * 16, 16)] = jnp.where(valid, idx16, 0)
    tibuf_v[pl.ds(j * 16, 16)] = jnp.where(valid, idx16, -1)
    return 0

  lax.fori_loop(0, _HALF // 16, idx_step, 0)

  base_out = e * _CAP + half * _HALF
  nfull = v // _CHUNK
  rem = v - nfull * _CHUNK

  def g_issue(c, buf, sem):
    pltpu.async_copy(x_hbm.at[idxc_v.at[pl.ds(c * _CHUNK, _CHUNK)]], buf, sem)

  def g_wait(buf, sem):
    pltpu.make_async_copy(x_hbm.at[idxc_v.at[pl.ds(0, _CHUNK)]],
                          buf, sem).wait()

  def w_issue(c, buf, sem):
    pltpu.async_copy(buf, out_x.at[pl.ds(base_out + c * _CHUNK, _CHUNK)], sem)

  def w_wait(buf, sem):
    pltpu.make_async_copy(buf, out_x.at[pl.ds(base_out, _CHUNK)], sem).wait()

  bufs = (buf0_v, buf1_v, buf2_v, buf3_v)
  gsems = (gsem0, gsem1, gsem2, gsem3)
  wsems = (wsem0, wsem1, wsem2, wsem3)

  for b in range(4):
    @pl.when(b < nfull)
    def _(b=b):
      g_issue(b, bufs[b], gsems[b])


  for g in range(_HALF // 128):
    pltpu.async_copy(ew_hbm.at[idxc_v.at[pl.ds(g * 128, 128)]],
                     wbuf_v.at[pl.ds(g * 128, 128)], esem)
  for g in range(_HALF // 128):
    pltpu.make_async_copy(ew_hbm.at[idxc_v.at[pl.ds(g * 128, 128)]],
                          wbuf_v.at[pl.ds(g * 128, 128)], esem).wait()

  def wmask_step(j, _):
    jglob = iota + j * 16
    w16 = wbuf_v[pl.ds(j * 16, 16)]
    wbuf_v[pl.ds(j * 16, 16)] = jnp.where(jglob < v, w16, 0.0)
    return 0

  lax.fori_loop(0, _HALF // 16, wmask_step, 0)

  pltpu.sync_copy(wbuf_v, out_w.at[pl.ds(base_out, _HALF)])
  pltpu.sync_copy(tibuf_v, out_ti.at[pl.ds(base_out, _HALF)])

  def group_step(gi, _):
    cbase = gi * 4
    for b in range(4):
      c = cbase + b

      @pl.when(c < nfull)
      def _(b=b, c=c):
        g_wait(bufs[b], gsems[b])
        w_issue(c, bufs[b], wsems[b])

        @pl.when(c + 4 < nfull)
        def _():
          w_wait(bufs[b], wsems[b])
          g_issue(c + 4, bufs[b], gsems[b])

      del c
    return 0

  ngroups = (nfull + 3) // 4
  lax.fori_loop(0, ngroups, group_step, 0)

  for b in range(4):
    @pl.when(b < nfull)
    def _(b=b):
      w_wait(bufs[b], wsems[b])

  @pl.when(rem > 0)
  def _():
    g_issue(nfull, buf0_v, gsem0)
    g_wait(buf0_v, gsem0)
    dst = base_out + nfull * _CHUNK

    def row_step(r, _):
      @pl.when(r < rem)
      def _():
        pltpu.async_copy(buf0_v.at[pl.ds(r, 1)],
                         out_x.at[pl.ds(dst + r, 1)], ssem)

      @pl.when(r >= rem)
      def _():
        pltpu.async_copy(zbuf_v.at[pl.ds(r, 1)],
                         out_x.at[pl.ds(dst + r, 1)], ssem)
      return 0

    lax.fori_loop(0, _CHUNK, row_step, 0)

  zstart = nfull + jnp.where(rem > 0, 1, 0).astype(jnp.int32)

  def zwave(wv, _):
    c = zstart + wv * 4
    for k in range(4):
      @pl.when(c + k < _NCHUNK)
      def _(k=k):
        pltpu.async_copy(zbuf_v,
                         out_x.at[pl.ds(base_out + (c + k) * _CHUNK, _CHUNK)],
                         zsem)
    for k in range(4):
      @pl.when(c + k < _NCHUNK)
      def _(k=k):
        pltpu.make_async_copy(zbuf_v,
                              out_x.at[pl.ds(base_out, _CHUNK)], zsem).wait()
    return 0

  lax.fori_loop(0, _NCHUNK // 4, zwave, 0)

  @pl.when(rem > 0)
  def _():
    def row_drain(r, _):
      pltpu.make_async_copy(zbuf_v.at[pl.ds(0, 1)],
                            out_x.at[pl.ds(base_out, 1)], ssem).wait()
      return 0

    lax.fori_loop(0, _CHUNK, row_drain, 0)


@jax.jit
def _dispatch(x, expert_ids, expert_weights):
  z = jnp.zeros((_CHUNK, _D), jnp.float32)
  mesh = plsc.VectorSubcoreMesh(core_axis_name="c", subcore_axis_name="s",
                                num_cores=2, num_subcores=16)
  fn = pl.kernel(
      _body,
      out_type=(
          jax.ShapeDtypeStruct((_E * _CAP, _D), jnp.float32),
          jax.ShapeDtypeStruct((_E * _CAP,), jnp.float32),
          jax.ShapeDtypeStruct((_E * _CAP,), jnp.int32),
          jax.ShapeDtypeStruct((16,), jnp.int32),
      ),
      mesh=mesh,
      compiler_params=pltpu.CompilerParams(needs_layout_passes=False),
      scratch_types=[
          pltpu.VMEM((_T,), jnp.int32),
          pltpu.VMEM((_LIST,), jnp.int32),
          pltpu.VMEM((_HALF,), jnp.int32),
          pltpu.VMEM((_CHUNK, _D), jnp.float32),
          pltpu.VMEM((_CHUNK, _D), jnp.float32),
          pltpu.VMEM((_CHUNK, _D), jnp.float32),
          pltpu.VMEM((_CHUNK, _D), jnp.float32),
          pltpu.VMEM((_CHUNK, _D), jnp.float32),
          pltpu.VMEM((_HALF,), jnp.float32),
          pltpu.VMEM((_HALF,), jnp.int32),
          pltpu.VMEM((16,), jnp.int32),
          pltpu.SMEM((1,), jnp.int32),
          pltpu.SemaphoreType.DMA,
          pltpu.SemaphoreType.DMA,
          pltpu.SemaphoreType.DMA,
          pltpu.SemaphoreType.DMA,
          pltpu.SemaphoreType.DMA,
          pltpu.SemaphoreType.DMA,
          pltpu.SemaphoreType.DMA,
          pltpu.SemaphoreType.DMA,
          pltpu.SemaphoreType.DMA,
          pltpu.SemaphoreType.DMA,
          pltpu.SemaphoreType.DMA,
      ],
  )
  return fn(x, expert_ids, expert_weights, z)


def kernel(x, expert_ids, expert_weights):
  out_x, out_w, out_ti, out_d = _dispatch(x, expert_ids, expert_weights)
  return (out_x.reshape(_E, _CAP, _D),
          out_w.reshape(_E, _CAP),
          out_ti.reshape(_E, _CAP),
          out_d[0])
